# baseline (speedup 1.0000x reference)
import functools

import jax
import jax.numpy as jnp
from jax import lax
from jax.experimental import pallas as pl
from jax.experimental.pallas import tpu as pltpu
from jax.experimental.pallas import tpu_sc as plsc

N_TOK = 8192
C_IN = 192
D = 64
K = 8192
TM = 512
KB = 1024


def _enorm_body(e_ref, en_ref):
    e = e_ref[...]
    en_ref[...] = jnp.sum(e * e, axis=1).reshape(1, K)


def _tc_body(z_ref, w_ref, b_ref, e_ref, en_ref, idx_ref, loss_ref):
    i = pl.program_id(0)
    zb = z_ref[...]
    ze = jnp.dot(zb, w_ref[...],
                 preferred_element_type=jnp.float32) + b_ref[...]
    fnorm = jnp.sum(ze * ze, axis=1, keepdims=True)
    zem2 = ze * (-2.0)

    best = jnp.full((TM, 1), jnp.inf, jnp.float32)
    bidxf = jnp.zeros((TM, 1), jnp.float32)
    iot = lax.broadcasted_iota(jnp.int32, (TM, KB), 1).astype(jnp.float32)
    for j in range(K // KB):
        ec = e_ref[pl.ds(j * KB, KB), :]
        s2 = lax.dot_general(zem2, ec, (((1,), (1,)), ((), ())),
                             preferred_element_type=jnp.float32)
        en = en_ref[:, pl.ds(j * KB, KB)]
        dch = (fnorm + s2) + en
        lmin = jnp.min(dch, axis=1, keepdims=True)
        lidx = jnp.min(jnp.where(dch == lmin, iot, float(K)),
                       axis=1, keepdims=True) + float(j * KB)
        take = lmin < best
        best = jnp.where(take, lmin, best)
        bidxf = jnp.where(take, lidx, bidxf)

    idx_ref[...] = bidxf.astype(jnp.int32).reshape(1, TM, 1)
    prev = jnp.where(i == 0, 0.0, loss_ref[...])
    loss_ref[...] = prev + jnp.sum(best).reshape(1, 1)


def _distance_argmin(z2, w_t, b2, embed):
    en = pl.pallas_call(
        _enorm_body,
        out_shape=jax.ShapeDtypeStruct((1, K), jnp.float32),
    )(embed)
    return pl.pallas_call(
        _tc_body,
        grid=(N_TOK // TM,),
        in_specs=[
            pl.BlockSpec((TM, C_IN), lambda i: (i, 0)),
            pl.BlockSpec((C_IN, D), lambda i: (0, 0)),
            pl.BlockSpec((1, D), lambda i: (0, 0)),
            pl.BlockSpec((K, D), lambda i: (0, 0)),
            pl.BlockSpec((1, K), lambda i: (0, 0)),
        ],
        out_specs=[
            pl.BlockSpec((1, TM, 1), lambda i: (i, 0, 0)),
            pl.BlockSpec((1, 1), lambda i: (0, 0)),
        ],
        out_shape=[
            jax.ShapeDtypeStruct((N_TOK // TM, TM, 1), jnp.int32),
            jax.ShapeDtypeStruct((1, 1), jnp.float32),
        ],
    )(z2, w_t, b2, embed, en)


@functools.cache
def _make_gather():
    info = plsc.get_sparse_core_info()
    nw = info.num_cores * info.num_subcores
    ch = 128
    rounds = N_TOK // (nw * ch)
    mesh = plsc.VectorSubcoreMesh(core_axis_name="c", subcore_axis_name="s")

    @functools.partial(
        pl.kernel, mesh=mesh,
        compiler_params=pltpu.CompilerParams(use_tc_tiling_on_sc=False),
        out_type=jax.ShapeDtypeStruct((N_TOK, D), jnp.float32),
        scratch_types=[
            pltpu.VMEM((ch,), jnp.int32),
            pltpu.VMEM((ch, D), jnp.float32),
            pltpu.SemaphoreType.DMA,
        ],
    )
    def gather(table_hbm, idx_hbm, out_hbm, idx_v, rows_v, sem):
        wid = lax.axis_index("s") * info.num_cores + lax.axis_index("c")
        for g in range(rounds):
            base = (g * nw + wid) * ch
            pltpu.sync_copy(idx_hbm.at[pl.ds(base, ch)], idx_v)
            pltpu.async_copy(table_hbm.at[idx_v], rows_v, sem).wait()
            pltpu.sync_copy(rows_v, out_hbm.at[pl.ds(base, ch)])

    return gather


def kernel(z, proj_w, proj_b, embed):
    B, C, H, W = z.shape
    z2 = z.transpose(0, 2, 3, 1).reshape(N_TOK, C_IN)
    idx_blk, loss_acc = _distance_argmin(
        z2, proj_w.T, proj_b.reshape(1, D), embed)
    idx = idx_blk.reshape(N_TOK)
    z_q = _make_gather()(embed, idx)
    z_q_flat = z_q.reshape(B, H, W, D)
    z_q_st = z_q_flat.transpose(0, 3, 1, 2)
    latent_loss = (loss_acc * (12.5 / (N_TOK * D))).reshape(())
    z_q_ind = idx.reshape(B, H, W)
    return (z_q_st, z_q_flat, latent_loss, z_q_ind)


# speedup vs baseline: 1.0311x; 1.0311x over previous
import functools

import jax
import jax.numpy as jnp
from jax import lax
from jax.experimental import pallas as pl
from jax.experimental.pallas import tpu as pltpu
from jax.experimental.pallas import tpu_sc as plsc

N_TOK = 8192
C_IN = 192
D = 64
K = 8192
TM = 512
KB = 1024


def _tc_body(z_ref, w_ref, b_ref, e_ref, idx_ref, loss_ref, en_ref):
    i = pl.program_id(0)

    @pl.when(i == 0)
    def _():
        e = e_ref[...]
        en_ref[...] = jnp.sum(e * e, axis=1).reshape(1, K)

    zb = z_ref[...]
    ze = jnp.dot(zb, w_ref[...],
                 preferred_element_type=jnp.float32) + b_ref[...]
    fnorm = jnp.sum(ze * ze, axis=1, keepdims=True)
    zem2 = ze * (-2.0)

    best = jnp.full((TM, 1), jnp.inf, jnp.float32)
    bidxf = jnp.zeros((TM, 1), jnp.float32)
    iot = lax.broadcasted_iota(jnp.int32, (TM, KB), 1).astype(jnp.float32)
    for j in range(K // KB):
        ec = e_ref[pl.ds(j * KB, KB), :]
        s2 = lax.dot_general(zem2, ec, (((1,), (1,)), ((), ())),
                             preferred_element_type=jnp.float32)
        en = en_ref[:, pl.ds(j * KB, KB)]
        dch = (fnorm + s2) + en
        lmin = jnp.min(dch, axis=1, keepdims=True)
        lidx = jnp.min(jnp.where(dch == lmin, iot, float(K)),
                       axis=1, keepdims=True) + float(j * KB)
        take = lmin < best
        best = jnp.where(take, lmin, best)
        bidxf = jnp.where(take, lidx, bidxf)

    idx_ref[...] = bidxf.astype(jnp.int32).reshape(1, TM, 1)
    prev = jnp.where(i == 0, 0.0, loss_ref[...])
    loss_ref[...] = prev + jnp.sum(best).reshape(1, 1)


def _distance_argmin(z2, w_t, b2, embed):
    return pl.pallas_call(
        _tc_body,
        grid=(N_TOK // TM,),
        in_specs=[
            pl.BlockSpec((TM, C_IN), lambda i: (i, 0)),
            pl.BlockSpec((C_IN, D), lambda i: (0, 0)),
            pl.BlockSpec((1, D), lambda i: (0, 0)),
            pl.BlockSpec((K, D), lambda i: (0, 0)),
        ],
        out_specs=[
            pl.BlockSpec((1, TM, 1), lambda i: (i, 0, 0)),
            pl.BlockSpec((1, 1), lambda i: (0, 0)),
        ],
        out_shape=[
            jax.ShapeDtypeStruct((N_TOK // TM, TM, 1), jnp.int32),
            jax.ShapeDtypeStruct((1, 1), jnp.float32),
        ],
        scratch_shapes=[pltpu.VMEM((1, K), jnp.float32)],
    )(z2, w_t, b2, embed)


@functools.cache
def _make_gather():
    info = plsc.get_sparse_core_info()
    nw = info.num_cores * info.num_subcores
    ch = 128
    rounds = N_TOK // (nw * ch)
    mesh = plsc.VectorSubcoreMesh(core_axis_name="c", subcore_axis_name="s")

    @functools.partial(
        pl.kernel, mesh=mesh,
        compiler_params=pltpu.CompilerParams(use_tc_tiling_on_sc=False),
        out_type=jax.ShapeDtypeStruct((N_TOK, D), jnp.float32),
        scratch_types=[
            pltpu.VMEM((ch,), jnp.int32),
            pltpu.VMEM((ch, D), jnp.float32),
            pltpu.SemaphoreType.DMA,
        ],
    )
    def gather(table_hbm, idx_hbm, out_hbm, idx_v, rows_v, sem):
        wid = lax.axis_index("s") * info.num_cores + lax.axis_index("c")
        for g in range(rounds):
            base = (g * nw + wid) * ch
            pltpu.sync_copy(idx_hbm.at[pl.ds(base, ch)], idx_v)
            pltpu.async_copy(table_hbm.at[idx_v], rows_v, sem).wait()
            pltpu.sync_copy(rows_v, out_hbm.at[pl.ds(base, ch)])

    return gather


def kernel(z, proj_w, proj_b, embed):
    B, C, H, W = z.shape
    z2 = z.transpose(0, 2, 3, 1).reshape(N_TOK, C_IN)
    idx_blk, loss_acc = _distance_argmin(
        z2, proj_w.T, proj_b.reshape(1, D), embed)
    idx = idx_blk.reshape(N_TOK)
    z_q = _make_gather()(embed, idx)
    z_q_flat = z_q.reshape(B, H, W, D)
    z_q_st = z_q_flat.transpose(0, 3, 1, 2)
    latent_loss = (loss_acc * (12.5 / (N_TOK * D))).reshape(())
    z_q_ind = idx.reshape(B, H, W)
    return (z_q_st, z_q_flat, latent_loss, z_q_ind)


# speedup vs baseline: 1.0445x; 1.0131x over previous
import functools

import jax
import jax.numpy as jnp
from jax import lax
from jax.experimental import pallas as pl
from jax.experimental.pallas import tpu as pltpu
from jax.experimental.pallas import tpu_sc as plsc

N_TOK = 8192
C_IN = 192
D = 64
K = 8192
TMT = 1024
KB = 1024


def _tc_body(z_ref, w_ref, b_ref, e_ref, idx_ref, loss_ref, en_ref):
    i = pl.program_id(0)

    @pl.when(i == 0)
    def _():
        e = e_ref[...]
        en_ref[...] = jnp.sum(e * e, axis=1, keepdims=True)

    zc = z_ref[0]
    ze = jnp.dot(w_ref[...], zc,
                 preferred_element_type=jnp.float32) + b_ref[...]
    fnorm = jnp.sum(ze * ze, axis=0, keepdims=True)
    zem2 = ze * (-2.0)

    best = jnp.full((1, TMT), jnp.inf, jnp.float32)
    bidxf = jnp.zeros((1, TMT), jnp.float32)
    iot = lax.broadcasted_iota(jnp.int32, (KB, TMT), 0).astype(jnp.float32)
    for j in range(K // KB):
        ec = e_ref[pl.ds(j * KB, KB), :]
        s2 = lax.dot_general(ec, zem2, (((1,), (0,)), ((), ())),
                             preferred_element_type=jnp.float32)
        en = en_ref[pl.ds(j * KB, KB), :]
        dch = (fnorm + s2) + en
        lmin = jnp.min(dch, axis=0, keepdims=True)
        lidx = jnp.min(jnp.where(dch == lmin, iot, float(K)),
                       axis=0, keepdims=True) + float(j * KB)
        take = lmin < best
        best = jnp.where(take, lmin, best)
        bidxf = jnp.where(take, lidx, bidxf)

    idx_ref[...] = bidxf.astype(jnp.int32).reshape(1, 1, TMT)
    prev = jnp.where(i == 0, 0.0, loss_ref[...])
    loss_ref[...] = prev + jnp.sum(best).reshape(1, 1)


def _distance_argmin(z3, w, b2, embed):
    return pl.pallas_call(
        _tc_body,
        grid=(N_TOK // TMT,),
        in_specs=[
            pl.BlockSpec((1, C_IN, TMT), lambda i: (i, 0, 0)),
            pl.BlockSpec((D, C_IN), lambda i: (0, 0)),
            pl.BlockSpec((D, 1), lambda i: (0, 0)),
            pl.BlockSpec((K, D), lambda i: (0, 0)),
        ],
        out_specs=[
            pl.BlockSpec((1, 1, TMT), lambda i: (i, 0, 0)),
            pl.BlockSpec((1, 1), lambda i: (0, 0)),
        ],
        out_shape=[
            jax.ShapeDtypeStruct((N_TOK // TMT, 1, TMT), jnp.int32),
            jax.ShapeDtypeStruct((1, 1), jnp.float32),
        ],
        scratch_shapes=[pltpu.VMEM((K, 1), jnp.float32)],
    )(z3, w, b2, embed)


@functools.cache
def _make_gather():
    info = plsc.get_sparse_core_info()
    nw = info.num_cores * info.num_subcores
    ch = 128
    rounds = N_TOK // (nw * ch)
    mesh = plsc.VectorSubcoreMesh(core_axis_name="c", subcore_axis_name="s")

    @functools.partial(
        pl.kernel, mesh=mesh,
        compiler_params=pltpu.CompilerParams(use_tc_tiling_on_sc=False),
        out_type=jax.ShapeDtypeStruct((N_TOK, D), jnp.float32),
        scratch_types=[
            pltpu.VMEM((ch,), jnp.int32),
            pltpu.VMEM((ch, D), jnp.float32),
            pltpu.SemaphoreType.DMA,
        ],
    )
    def gather(table_hbm, idx_hbm, out_hbm, idx_v, rows_v, sem):
        wid = lax.axis_index("s") * info.num_cores + lax.axis_index("c")
        for g in range(rounds):
            base = (g * nw + wid) * ch
            pltpu.sync_copy(idx_hbm.at[pl.ds(base, ch)], idx_v)
            pltpu.async_copy(table_hbm.at[idx_v], rows_v, sem).wait()
            pltpu.sync_copy(rows_v, out_hbm.at[pl.ds(base, ch)])

    return gather


def kernel(z, proj_w, proj_b, embed):
    B, C, H, W = z.shape
    z3 = z.reshape(B, C_IN, H * W)
    idx_blk, loss_acc = _distance_argmin(
        z3, proj_w, proj_b.reshape(D, 1), embed)
    idx = idx_blk.reshape(N_TOK)
    z_q = _make_gather()(embed, idx)
    z_q_flat = z_q.reshape(B, H, W, D)
    z_q_st = z_q_flat.transpose(0, 3, 1, 2)
    latent_loss = (loss_acc * (12.5 / (N_TOK * D))).reshape(())
    z_q_ind = idx.reshape(B, H, W)
    return (z_q_st, z_q_flat, latent_loss, z_q_ind)
